# baseline (speedup 1.0000x reference)
import jax
import jax.numpy as jnp
from jax import lax
from jax.experimental import pallas as pl
from jax.experimental.pallas import tpu as pltpu
from jax.experimental.pallas import tpu_sc as plsc

N = 10000
D = 128
E = 320000

NC = 2
NS = 16
NW = NC * NS
CHUNK = 80
NCHUNKS = 128
EDGES_PER_TILE = NCHUNKS * CHUNK
NT = 10016
NPAD = NS * 640
ROWS_PER_TILE = NPAD // NS

_MESH = plsc.VectorSubcoreMesh(core_axis_name="c", subcore_axis_name="s")


def _sc_pass(table, srcf, dstf, zeros_nd, zeros_n, ones_c, with_deg):
  out_type = [jax.ShapeDtypeStruct((NC, NPAD, D), jnp.float32)]
  scratch = [
      pltpu.VMEM_SHARED((NPAD, D), jnp.float32),
      [pltpu.VMEM((CHUNK,), jnp.int32) for _ in range(8)],
      [pltpu.VMEM((CHUNK,), jnp.int32) for _ in range(8)],
      [pltpu.SemaphoreType.DMA for _ in range(8)],
      [pltpu.VMEM((CHUNK, D), jnp.float32) for _ in range(4)],
      [pltpu.SemaphoreType.DMA for _ in range(4)],
  ]
  if with_deg:
    out_type.append(jax.ShapeDtypeStruct((NC, NPAD), jnp.float32))
    scratch.append(pltpu.VMEM_SHARED((NPAD,), jnp.float32))
    scratch.append(pltpu.VMEM((CHUNK,), jnp.float32))

  def body(table_hbm, src_hbm, dst_hbm, znd_hbm, zn_hbm, ones_hbm,
           *outs_and_scratch):
    if with_deg:
      (out_h, out_deg, acc, ivs, dvs, isems, rowbufs, gsems,
       deg, ones_v) = outs_and_scratch
    else:
      out_h, acc, ivs, dvs, isems, rowbufs, gsems = outs_and_scratch
    c = lax.axis_index("c")
    s = lax.axis_index("s")
    wid = c * NS + s
    rows = pl.ds(s * ROWS_PER_TILE, ROWS_PER_TILE)

    pltpu.sync_copy(znd_hbm.at[rows], acc.at[rows])
    if with_deg:
      pltpu.sync_copy(zn_hbm.at[rows], deg.at[rows])
      pltpu.sync_copy(ones_hbm, ones_v)
    plsc.subcore_barrier()

    base = wid * EDGES_PER_TILE

    def iload(k, b):
      off = base + k * CHUNK
      pltpu.async_copy(src_hbm.at[pl.ds(off, CHUNK)], ivs[b], isems[b])
      pltpu.async_copy(dst_hbm.at[pl.ds(off, CHUNK)], dvs[b], isems[b])

    def iwait(b):
      pltpu.make_async_copy(src_hbm.at[pl.ds(0, CHUNK)], ivs[b],
                            isems[b]).wait()
      pltpu.make_async_copy(dst_hbm.at[pl.ds(0, CHUNK)], dvs[b],
                            isems[b]).wait()

    def gstart(b, r):
      pltpu.async_copy(table_hbm.at[ivs[b]], rowbufs[r], gsems[r])

    def gwait(b, r):
      pltpu.make_async_copy(table_hbm.at[ivs[b]], rowbufs[r],
                            gsems[r]).wait()

    def scatter(b, r):
      pltpu.sync_copy(rowbufs[r], acc.at[dvs[b]], add=True)
      if with_deg:
        pltpu.sync_copy(ones_v, deg.at[dvs[b]], add=True)

    for k in range(5):
      iload(k, k)
    for k in range(3):
      iwait(k)
      gstart(k, k)

    def step(kk, carry):
      c0 = 8 * kk
      for j in range(8):
        c = c0 + j
        gwait(j % 8, j % 4)

        @pl.when(c + 3 <= NCHUNKS - 1)
        def _():
          iwait((j + 3) % 8)
          gstart((j + 3) % 8, (j + 3) % 4)

        @pl.when(c + 5 <= NCHUNKS - 1)
        def _():
          iload(c + 5, (j + 5) % 8)

        scatter(j % 8, j % 4)
      return carry

    lax.fori_loop(0, NCHUNKS // 8, step, 0)
    plsc.subcore_barrier()

    pltpu.sync_copy(acc.at[rows], out_h.at[c, rows])
    if with_deg:
      pltpu.sync_copy(deg.at[rows], out_deg.at[c, rows])

  fn = pl.kernel(body, out_type=out_type, mesh=_MESH, scratch_types=scratch)
  return fn(table, srcf, dstf, zeros_nd, zeros_n, ones_c)


def _combine_body(pa_ref, pd_ref, out_ref):
  total = pa_ref[0] + pa_ref[1]
  deg = pd_ref[0] + pd_ref[1]
  inv = 1.0 / jnp.maximum(deg, 1.0)
  out_ref[...] = total * inv


_ROWB = 1000


def _combine(pa, pd3):
  grid = (N // _ROWB,)
  return pl.pallas_call(
      _combine_body,
      grid=grid,
      in_specs=[
          pl.BlockSpec((NC, _ROWB, D), lambda i: (0, i, 0)),
          pl.BlockSpec((NC, _ROWB, 1), lambda i: (0, i, 0)),
      ],
      out_specs=pl.BlockSpec((_ROWB, D), lambda i: (i, 0)),
      out_shape=jax.ShapeDtypeStruct((N, D), jnp.float32),
  )(pa, pd3)


def kernel(x, edge_index):
  ei = edge_index.astype(jnp.int32)
  ei3 = ei.reshape(2, NW, E // NW)
  ei3 = jnp.pad(ei3, ((0, 0), (0, 0), (0, EDGES_PER_TILE - E // NW)),
                constant_values=N)
  srcf = ei3[0].reshape(-1)
  dstf = ei3[1].reshape(-1)
  xp = jnp.pad(x, ((0, NT - N), (0, 0)))
  zeros_nd = jnp.zeros((NPAD, D), jnp.float32)
  zeros_n = jnp.zeros((NPAD,), jnp.float32)
  ones_c = jnp.ones((CHUNK,), jnp.float32)

  ph, pdeg = _sc_pass(xp, srcf, dstf, zeros_nd, zeros_n, ones_c, with_deg=True)
  pd3 = pdeg[:, :, None]
  h = _combine(ph, pd3)
  hp = jnp.pad(h, ((0, NT - N), (0, 0)))
  (ph2,) = _sc_pass(hp, srcf, dstf, zeros_nd, zeros_n, ones_c, with_deg=False)
  return _combine(ph2, pd3)


# speedup vs baseline: 3.1621x; 3.1621x over previous
import jax
import jax.numpy as jnp
from jax import lax
from jax.experimental import pallas as pl
from jax.experimental.pallas import tpu as pltpu
from jax.experimental.pallas import tpu_sc as plsc

N = 10000
D = 128
E = 320000

NC = 2
NS = 16
NW = NC * NS
EDGES_PER_TILE = E // NW
CHUNK = 80
NCHUNKS = EDGES_PER_TILE // CHUNK
NPAD = NS * 640
ROWS_PER_TILE = NPAD // NS

_MESH = plsc.VectorSubcoreMesh(core_axis_name="c", subcore_axis_name="s")


def _sc_pass(table, srcf, dstf, zeros_nd, zeros_n, ones_c, with_deg):
  out_type = [jax.ShapeDtypeStruct((NC, NPAD, D), jnp.float32)]
  scratch = [
      pltpu.VMEM_SHARED((NPAD, D), jnp.float32),
      [pltpu.VMEM((CHUNK,), jnp.int32) for _ in range(4)],
      [pltpu.VMEM((CHUNK,), jnp.int32) for _ in range(4)],
      [pltpu.SemaphoreType.DMA for _ in range(4)],
      [pltpu.VMEM((CHUNK, D), jnp.float32) for _ in range(4)],
      [pltpu.SemaphoreType.DMA for _ in range(4)],
  ]
  if with_deg:
    out_type.append(jax.ShapeDtypeStruct((NC, NPAD), jnp.float32))
    scratch.append(pltpu.VMEM_SHARED((NPAD,), jnp.float32))
    scratch.append(pltpu.VMEM((CHUNK,), jnp.float32))

  def body(table_hbm, src_hbm, dst_hbm, znd_hbm, zn_hbm, ones_hbm,
           *outs_and_scratch):
    if with_deg:
      (out_h, out_deg, acc, ivs, dvs, isems, rowbufs, gsems,
       deg, ones_v) = outs_and_scratch
    else:
      out_h, acc, ivs, dvs, isems, rowbufs, gsems = outs_and_scratch
    c = lax.axis_index("c")
    s = lax.axis_index("s")
    wid = c * NS + s
    rows = pl.ds(s * ROWS_PER_TILE, ROWS_PER_TILE)

    pltpu.sync_copy(znd_hbm.at[rows], acc.at[rows])
    if with_deg:
      pltpu.sync_copy(zn_hbm.at[rows], deg.at[rows])
      pltpu.sync_copy(ones_hbm, ones_v)
    plsc.subcore_barrier()

    base = wid * EDGES_PER_TILE

    def iload(k, b):
      off = base + k * CHUNK
      pltpu.async_copy(src_hbm.at[pl.ds(off, CHUNK)], ivs[b], isems[b])
      pltpu.async_copy(dst_hbm.at[pl.ds(off, CHUNK)], dvs[b], isems[b])

    def iwait(b):
      pltpu.make_async_copy(src_hbm.at[pl.ds(0, CHUNK)], ivs[b],
                            isems[b]).wait()
      pltpu.make_async_copy(dst_hbm.at[pl.ds(0, CHUNK)], dvs[b],
                            isems[b]).wait()

    def gstart(b, r):
      pltpu.async_copy(table_hbm.at[ivs[b]], rowbufs[r], gsems[r])

    def gwait(b, r):
      pltpu.make_async_copy(table_hbm.at[ivs[b]], rowbufs[r],
                            gsems[r]).wait()

    def scatter(b, r):
      pltpu.sync_copy(rowbufs[r], acc.at[dvs[b]], add=True)
      if with_deg:
        pltpu.sync_copy(ones_v, deg.at[dvs[b]], add=True)

    iload(0, 0)
    iload(1, 1)
    iload(2, 2)
    iwait(0)
    gstart(0, 0)
    iwait(1)
    gstart(1, 1)

    def step(kk, carry):
      c0 = 4 * kk
      for j in range(4):
        c = c0 + j
        gwait(j, j)

        @pl.when(c + 2 <= NCHUNKS - 1)
        def _():
          iwait((j + 2) % 4)
          gstart((j + 2) % 4, (j + 2) % 4)

        @pl.when(c + 3 <= NCHUNKS - 1)
        def _():
          iload(c + 3, (j + 3) % 4)

        scatter(j, j)
      return carry

    lax.fori_loop(0, (NCHUNKS - 1) // 4, step, 0)

    gwait(0, 0)
    scatter(0, 0)
    plsc.subcore_barrier()

    pltpu.sync_copy(acc.at[rows], out_h.at[c, rows])
    if with_deg:
      pltpu.sync_copy(deg.at[rows], out_deg.at[c, rows])

  fn = pl.kernel(body, out_type=out_type, mesh=_MESH, scratch_types=scratch)
  return fn(table, srcf, dstf, zeros_nd, zeros_n, ones_c)


def _combine_body(pa_ref, pd_ref, out_ref):
  total = pa_ref[0] + pa_ref[1]
  deg = pd_ref[0] + pd_ref[1]
  inv = 1.0 / jnp.maximum(deg, 1.0)
  out_ref[...] = total * inv


_ROWB = 1000


def _combine(pa, pd3):
  grid = (N // _ROWB,)
  return pl.pallas_call(
      _combine_body,
      grid=grid,
      in_specs=[
          pl.BlockSpec((NC, _ROWB, D), lambda i: (0, i, 0)),
          pl.BlockSpec((NC, _ROWB, 1), lambda i: (0, i, 0)),
      ],
      out_specs=pl.BlockSpec((_ROWB, D), lambda i: (i, 0)),
      out_shape=jax.ShapeDtypeStruct((N, D), jnp.float32),
  )(pa, pd3)


def kernel(x, edge_index):
  ei = edge_index.astype(jnp.int32)
  srcf = ei[0]
  dstf = ei[1]
  zeros_nd = jnp.zeros((NPAD, D), jnp.float32)
  zeros_n = jnp.zeros((NPAD,), jnp.float32)
  ones_c = jnp.ones((CHUNK,), jnp.float32)

  ph, pdeg = _sc_pass(x, srcf, dstf, zeros_nd, zeros_n, ones_c, with_deg=True)
  pd3 = pdeg[:, :, None]
  h = _combine(ph, pd3)
  (ph2,) = _sc_pass(h, srcf, dstf, zeros_nd, zeros_n, ones_c, with_deg=False)
  return _combine(ph2, pd3)
